# baseline (speedup 1.0000x reference)
import functools

import jax
import jax.numpy as jnp
from jax import lax
from jax.experimental import pallas as pl
from jax.experimental.pallas import tpu as pltpu
from jax.experimental.pallas import tpu_sc as plsc

B = 16384
D = 32
L = 16
NC = 2
NS = 16
NW = NC * NS
BPW = B // NW
CHUNK = 128
NCHUNK = BPW // CHUNK

_mesh = plsc.VectorSubcoreMesh(core_axis_name="c", subcore_axis_name="s")


@functools.partial(
    pl.kernel,
    mesh=_mesh,
    compiler_params=pltpu.CompilerParams(use_tc_tiling_on_sc=False),
    out_type=jax.ShapeDtypeStruct((B,), jnp.float32),
    scratch_types=[
        pltpu.VMEM((NCHUNK, CHUNK), jnp.int32),
        pltpu.VMEM((NCHUNK, CHUNK), jnp.int32),
        pltpu.VMEM((2, D, CHUNK), jnp.float32),
        pltpu.VMEM((2, D, CHUNK), jnp.float32),
        pltpu.VMEM((BPW,), jnp.float32),
        pltpu.SemaphoreType.DMA,
    ],
)
def _w2v_kernel(cw_hbm, xw_hbm, ctab_hbm, xtab_hbm, out_hbm,
                cw_v, xw_v, cp_v, xp_v, o_v, sem):
    wid = lax.axis_index("s") * NC + lax.axis_index("c")
    base_chunk = wid * NCHUNK

    pltpu.sync_copy(cw_hbm.at[pl.ds(base_chunk, NCHUNK)], cw_v)
    pltpu.sync_copy(xw_hbm.at[pl.ds(base_chunk, NCHUNK)], xw_v)

    def fetch(c, buf):
        cps = []
        for d in range(D):
            cps.append(pltpu.async_copy(
                ctab_hbm.at[d].at[cw_v.at[c]], cp_v.at[buf, d], sem))
            cps.append(pltpu.async_copy(
                xtab_hbm.at[d].at[xw_v.at[c]], xp_v.at[buf, d], sem))
        return cps

    pend = fetch(0, 0)
    for c in range(NCHUNK):
        for cp in pend:
            cp.wait()
        if c + 1 < NCHUNK:
            pend = fetch(c + 1, (c + 1) % 2)
        buf = c % 2

        for g in range(CHUNK // L):
            sl = pl.ds(g * L, L)
            acc = jnp.zeros((L,), jnp.float32)
            for d in range(D):
                acc = acc + cp_v[buf, d, sl] * xp_v[buf, d, sl]
            o_v[pl.ds(c * CHUNK + g * L, L)] = 1.0 / (1.0 + jnp.exp(-acc))

    pltpu.sync_copy(o_v, out_hbm.at[pl.ds(wid * BPW, BPW)])


def kernel(center_word, context_word, center_table, context_table):
    cw = center_word.astype(jnp.int32).reshape(B // CHUNK, CHUNK)
    xw = context_word.astype(jnp.int32).reshape(B // CHUNK, CHUNK)
    ct = center_table.T
    xt = context_table.T
    return _w2v_kernel(cw, xw, ct, xt)


# speedup vs baseline: 1.4527x; 1.4527x over previous
import functools

import jax
import jax.numpy as jnp
from jax import lax
from jax.experimental import pallas as pl
from jax.experimental.pallas import tpu as pltpu
from jax.experimental.pallas import tpu_sc as plsc

B = 16384
D = 32
L = 16
NC = 2
NS = 16
NW = NC * NS
BPW = B // NW
CHUNK = 128
NCHUNK = BPW // CHUNK

_mesh = plsc.VectorSubcoreMesh(core_axis_name="c", subcore_axis_name="s")


@functools.partial(
    pl.kernel,
    mesh=_mesh,
    compiler_params=pltpu.CompilerParams(use_tc_tiling_on_sc=False),
    out_type=jax.ShapeDtypeStruct((B,), jnp.float32),
    scratch_types=[
        pltpu.VMEM((NCHUNK, CHUNK), jnp.int32),
        pltpu.VMEM((NCHUNK, CHUNK), jnp.int32),
        pltpu.VMEM((2, D, CHUNK), jnp.float32),
        pltpu.VMEM((2, D, CHUNK), jnp.float32),
        pltpu.VMEM((BPW,), jnp.float32),
        pltpu.SemaphoreType.DMA,
    ],
)
def _w2v_kernel(cw_hbm, xw_hbm, ctab_hbm, xtab_hbm, out_hbm,
                cw_v, xw_v, cp_v, xp_v, o_v, sem):
    wid = lax.axis_index("s") * NC + lax.axis_index("c")
    base_chunk = wid * NCHUNK

    pltpu.sync_copy(cw_hbm.at[pl.ds(base_chunk, NCHUNK)], cw_v)
    pltpu.sync_copy(xw_hbm.at[pl.ds(base_chunk, NCHUNK)], xw_v)

    def fetch(c, buf):
        cps = []
        for d in range(D):
            cps.append(pltpu.async_copy(
                ctab_hbm.at[d].at[cw_v.at[c]], cp_v.at[buf, d], sem))
            cps.append(pltpu.async_copy(
                xtab_hbm.at[d].at[xw_v.at[c]], xp_v.at[buf, d], sem))
        return cps

    pend = fetch(0, 0)
    for c in range(NCHUNK):
        for cp in pend:
            cp.wait()
        if c + 1 < NCHUNK:
            pend = fetch(c + 1, (c + 1) % 2)
        buf = c % 2

        for g in range(CHUNK // L):
            sl = pl.ds(g * L, L)
            acc = jnp.zeros((L,), jnp.float32)
            for d in range(D):
                acc = acc + cp_v[buf, d, sl] * xp_v[buf, d, sl]
            o_v[pl.ds(c * CHUNK + g * L, L)] = 1.0 / (1.0 + jnp.exp(-acc))

    pltpu.sync_copy(o_v, out_hbm.at[pl.ds(wid * BPW, BPW)])


def kernel(center_word, context_word, center_table, context_table):
    cw = center_word.astype(jnp.int32).reshape(B // CHUNK, CHUNK)
    xw = context_word.astype(jnp.int32).reshape(B // CHUNK, CHUNK)
    ct = jnp.concatenate([center_table[:, d] for d in range(D)]).reshape(D, -1)
    xt = jnp.concatenate([context_table[:, d] for d in range(D)]).reshape(D, -1)
    return _w2v_kernel(cw, xw, ct, xt)


# speedup vs baseline: 5.7273x; 3.9424x over previous
import functools

import jax
import jax.numpy as jnp
from jax import lax
from jax.experimental import pallas as pl
from jax.experimental.pallas import tpu as pltpu
from jax.experimental.pallas import tpu_sc as plsc

B = 16384
D = 32
L = 16
NC = 2
NS = 16
NW = NC * NS
BPW = B // NW
CHUNK = 128
NCHUNK = BPW // CHUNK

_mesh = plsc.VectorSubcoreMesh(core_axis_name="c", subcore_axis_name="s")


@functools.partial(
    pl.kernel,
    mesh=_mesh,
    compiler_params=pltpu.CompilerParams(use_tc_tiling_on_sc=False),
    out_type=jax.ShapeDtypeStruct((B,), jnp.float32),
    scratch_types=[
        pltpu.VMEM((NCHUNK, CHUNK), jnp.int32),
        pltpu.VMEM((NCHUNK, CHUNK), jnp.int32),
        pltpu.VMEM((BPW, D), jnp.float32),
        pltpu.VMEM((BPW, D), jnp.float32),
        pltpu.VMEM((BPW,), jnp.float32),
        pltpu.SemaphoreType.DMA,
    ],
)
def _w2v_kernel(cw_hbm, xw_hbm, ctab_hbm, xtab_hbm, out_hbm,
                cw_v, xw_v, cr_v, xr_v, o_v, sem):
    wid = lax.axis_index("s") * NC + lax.axis_index("c")
    base_chunk = wid * NCHUNK

    pltpu.sync_copy(cw_hbm.at[pl.ds(base_chunk, NCHUNK)], cw_v)
    pltpu.sync_copy(xw_hbm.at[pl.ds(base_chunk, NCHUNK)], xw_v)

    copies = []
    for j in range(NCHUNK):
        dst = pl.ds(j * CHUNK, CHUNK)
        copies.append(pltpu.async_copy(ctab_hbm.at[cw_v.at[j]], cr_v.at[dst], sem))
        copies.append(pltpu.async_copy(xtab_hbm.at[xw_v.at[j]], xr_v.at[dst], sem))
    for c in copies:
        c.wait()

    lane = lax.iota(jnp.int32, L)
    perms = [lane ^ k for k in (8, 4, 2, 1)]

    def hsum(v):
        for p in perms:
            v = v + v.at[p].get(mode="promise_in_bounds")
        return v

    def body(g, carry):
        base = g * L
        out = jnp.zeros((L,), jnp.float32)
        for i in range(L):
            j = base + i
            c0 = cr_v[j, pl.ds(0, L)]
            c1 = cr_v[j, pl.ds(L, L)]
            x0 = xr_v[j, pl.ds(0, L)]
            x1 = xr_v[j, pl.ds(L, L)]
            s = c0 * x0 + c1 * x1
            out = jnp.where(lane == i, hsum(s), out)
        o_v[pl.ds(base, L)] = 1.0 / (1.0 + jnp.exp(-out))
        return carry

    lax.fori_loop(0, BPW // L, body, 0)

    pltpu.sync_copy(o_v, out_hbm.at[pl.ds(wid * BPW, BPW)])


def kernel(center_word, context_word, center_table, context_table):
    cw = center_word.astype(jnp.int32).reshape(B // CHUNK, CHUNK)
    xw = context_word.astype(jnp.int32).reshape(B // CHUNK, CHUNK)
    return _w2v_kernel(cw, xw, center_table, context_table)
